# Initial kernel scaffold; baseline (speedup 1.0000x reference)
import jax
import jax.numpy as jnp
from jax.experimental import pallas as pl


def kernel(top_indices, speaker_map, genre_id, genre_emb, distance_emb, speaker_emb):
    raise NotImplementedError("write your pallas kernel here")


# speedup vs baseline: 25.4592x; 25.4592x over previous
import jax
import jax.numpy as jnp
from jax import lax
from jax.experimental import pallas as pl
from jax.experimental.pallas import tpu as pltpu
from jax.experimental.pallas import tpu_sc as plsc

N_WORDS = 8192
K_ANT = 64
EMB = 32
M = N_WORDS * K_ANT
NC, NS = 2, 16
NW = NC * NS
CHUNK = M // NW
LANES = 16


def _sc_codes_body(top_hbm, spk_hbm, c_hbm, spk_v, top_v, c_v):
    wid = lax.axis_index("s") * NC + lax.axis_index("c")
    base = wid * CHUNK
    pltpu.sync_copy(spk_hbm, spk_v)
    pltpu.sync_copy(top_hbm.at[pl.ds(base, CHUNK)], top_v)

    def body(k, carry):
        off = k * LANES
        t = top_v[pl.ds(off, LANES)]
        p = base + off + lax.iota(jnp.int32, LANES)
        i = lax.shift_right_logical(p, 6)
        s_t = plsc.load_gather(spk_v, [t])
        s_i = plsc.load_gather(spk_v, [i])
        ss = (s_t == s_i).astype(jnp.int32)
        dist = jnp.maximum(i - t, 1)
        e = lax.shift_right_logical(
            lax.bitcast_convert_type(dist.astype(jnp.float32), jnp.int32), 23) - 127
        di = jnp.where(dist < 5, dist - 1, jnp.minimum(e, 6) + 2)
        c_v[pl.ds(off, LANES)] = ss * 9 + di
        return carry

    lax.fori_loop(0, CHUNK // LANES, body, 0)
    pltpu.sync_copy(c_v, c_hbm.at[pl.ds(base, CHUNK)])


_SC_CODES_CACHE = []


def _sc_codes():
    if not _SC_CODES_CACHE:
        _SC_CODES_CACHE.append(pl.kernel(
            _sc_codes_body,
            out_type=jax.ShapeDtypeStruct((M,), jnp.int32),
            mesh=plsc.VectorSubcoreMesh(
                core_axis_name="c", subcore_axis_name="s",
                num_cores=NC, num_subcores=NS),
            compiler_params=pltpu.CompilerParams(needs_layout_passes=False),
            scratch_types=[
                pltpu.VMEM((N_WORDS,), jnp.int32),
                pltpu.VMEM((CHUNK,), jnp.int32),
                pltpu.VMEM((CHUNK,), jnp.int32),
            ],
        ))
    return _SC_CODES_CACHE[0]

BP = 8192
GRID = M // BP


def _expand_body(c_ref, gid_ref, g_ref, d_ref, s_ref, out_ref):
    gid = gid_ref[...]
    g_row = jnp.zeros((1, EMB), jnp.float32)
    for k in range(7):
        g_row = g_row + jnp.where(gid == k, g_ref[k:k + 1, :], 0.0)
    row = lax.broadcasted_iota(jnp.int32, (32, 1), 0)
    spk_part = jnp.where(row < 9, s_ref[0:1, :], s_ref[1:2, :])
    dist_part = jnp.concatenate(
        [d_ref[...], d_ref[...], jnp.zeros((14, EMB), jnp.float32)], axis=0)
    genre_part = jnp.broadcast_to(g_row, (32, EMB))
    tab = jnp.concatenate([spk_part, dist_part, genre_part], axis=1)
    cb = c_ref[...]
    oh = (cb == lax.broadcasted_iota(jnp.int32, (BP, 32), 1)).astype(jnp.float32)
    out_ref[...] = jnp.dot(oh, tab, preferred_element_type=jnp.float32)


_expand = pl.pallas_call(
    _expand_body,
    grid=(GRID,),
    in_specs=[
        pl.BlockSpec((BP, 1), lambda b: (b, 0)),
        pl.BlockSpec((1, 1), lambda b: (0, 0)),
        pl.BlockSpec((7, EMB), lambda b: (0, 0)),
        pl.BlockSpec((9, EMB), lambda b: (0, 0)),
        pl.BlockSpec((2, EMB), lambda b: (0, 0)),
    ],
    out_specs=pl.BlockSpec((BP, 96), lambda b: (b, 0)),
    out_shape=jax.ShapeDtypeStruct((M, 96), jnp.float32),
)


def kernel(top_indices, speaker_map, genre_id, genre_emb, distance_emb, speaker_emb):
    top = top_indices.astype(jnp.int32).reshape(M)
    spk = speaker_map.astype(jnp.int32)
    c = _sc_codes()(top, spk)
    gid = jnp.asarray(genre_id, jnp.int32).reshape(1, 1)
    out = _expand(c.reshape(M, 1), gid,
                  genre_emb.astype(jnp.float32),
                  distance_emb.astype(jnp.float32),
                  speaker_emb.astype(jnp.float32))
    return out.reshape(N_WORDS, K_ANT, 96)


# speedup vs baseline: 40.4934x; 1.5905x over previous
import jax
import jax.numpy as jnp
from jax import lax
from jax.experimental import pallas as pl
from jax.experimental.pallas import tpu as pltpu
from jax.experimental.pallas import tpu_sc as plsc

N_WORDS = 8192
K_ANT = 64
EMB = 32
M = N_WORDS * K_ANT
NC, NS = 2, 16
NW = NC * NS
CHUNK = M // NW
LANES = 16


def _sc_codes_body(top_hbm, spk_hbm, c_hbm, spk_v, top_v, c_v):
    wid = lax.axis_index("s") * NC + lax.axis_index("c")
    base = wid * CHUNK
    pltpu.sync_copy(spk_hbm, spk_v)
    pltpu.sync_copy(top_hbm.at[pl.ds(base, CHUNK)], top_v)

    def body(k, carry):
        off = k * LANES
        t = top_v[pl.ds(off, LANES)]
        p = base + off + lax.iota(jnp.int32, LANES)
        i = lax.shift_right_logical(p, 6)
        s_t = plsc.load_gather(spk_v, [t])
        s_i = plsc.load_gather(spk_v, [i])
        ss = (s_t == s_i).astype(jnp.int32)
        dist = jnp.maximum(i - t, 1)
        e = lax.shift_right_logical(
            lax.bitcast_convert_type(dist.astype(jnp.float32), jnp.int32), 23) - 127
        di = jnp.where(dist < 5, dist - 1, jnp.minimum(e, 6) + 2)
        c_v[pl.ds(off, LANES)] = ss * 9 + di
        return carry

    lax.fori_loop(0, CHUNK // LANES, body, 0)
    pltpu.sync_copy(c_v, c_hbm.at[pl.ds(base, CHUNK)])


_SC_CODES_CACHE = []


def _sc_codes():
    if not _SC_CODES_CACHE:
        _SC_CODES_CACHE.append(pl.kernel(
            _sc_codes_body,
            out_type=jax.ShapeDtypeStruct((M,), jnp.int32),
            mesh=plsc.VectorSubcoreMesh(
                core_axis_name="c", subcore_axis_name="s",
                num_cores=NC, num_subcores=NS),
            compiler_params=pltpu.CompilerParams(needs_layout_passes=False),
            scratch_types=[
                pltpu.VMEM((N_WORDS,), jnp.int32),
                pltpu.VMEM((CHUNK,), jnp.int32),
                pltpu.VMEM((CHUNK,), jnp.int32),
            ],
        ))
    return _SC_CODES_CACHE[0]

BP = 8192
GRID = M // BP


BR = BP // 128


def _expand_body(c_ref, gid_ref, g_ref, d_ref, s_ref, out_ref):
    gid = gid_ref[...]
    g_row = jnp.zeros((1, EMB), jnp.float32)
    for k in range(7):
        g_row = g_row + jnp.where(gid == k, g_ref[k:k + 1, :], 0.0)
    row = lax.broadcasted_iota(jnp.int32, (32, 1), 0)
    spk_part = jnp.where(row < 9, s_ref[0:1, :], s_ref[1:2, :])
    dist_part = jnp.concatenate(
        [d_ref[...], d_ref[...], jnp.zeros((14, EMB), jnp.float32)], axis=0)
    genre_part = jnp.broadcast_to(g_row, (32, EMB))
    tab = jnp.concatenate([spk_part, dist_part, genre_part], axis=1)
    cb = c_ref[...]
    oh3 = (cb[:, :, None] ==
           lax.broadcasted_iota(jnp.int32, (BR, 128, 32), 2)).astype(jnp.float32)
    oh = jnp.reshape(oh3, (BP, 32))
    out_ref[...] = jnp.dot(oh, tab, preferred_element_type=jnp.float32)


_expand = pl.pallas_call(
    _expand_body,
    grid=(GRID,),
    in_specs=[
        pl.BlockSpec((BR, 128), lambda b: (b, 0)),
        pl.BlockSpec((1, 1), lambda b: (0, 0)),
        pl.BlockSpec((7, EMB), lambda b: (0, 0)),
        pl.BlockSpec((9, EMB), lambda b: (0, 0)),
        pl.BlockSpec((2, EMB), lambda b: (0, 0)),
    ],
    out_specs=pl.BlockSpec((BP, 96), lambda b: (b, 0)),
    out_shape=jax.ShapeDtypeStruct((M, 96), jnp.float32),
)


def kernel(top_indices, speaker_map, genre_id, genre_emb, distance_emb, speaker_emb):
    top = top_indices.astype(jnp.int32).reshape(M)
    spk = speaker_map.astype(jnp.int32)
    c = _sc_codes()(top, spk)
    gid = jnp.asarray(genre_id, jnp.int32).reshape(1, 1)
    out = _expand(c.reshape(M // 128, 128), gid,
                  genre_emb.astype(jnp.float32),
                  distance_emb.astype(jnp.float32),
                  speaker_emb.astype(jnp.float32))
    return out.reshape(N_WORDS, K_ANT, 96)
